# baseline (device time: 11617 ns/iter reference)
import os

import jax
import jax.numpy as jnp
from jax import lax
from jax.experimental import pallas as pl
from jax.experimental.pallas import tpu as pltpu

_MODE = os.environ.get("KERNEL_MODE", "")
if not _MODE:
    try:
        _MODE = (
            open(os.path.join(os.path.dirname(__file__), "kernel_mode.txt"))
            .read()
            .strip()
        )
    except OSError:
        _MODE = ""
_MODE = _MODE or "full"


def _read_int_file(name, default):
    try:
        return int(
            open(os.path.join(os.path.dirname(__file__), name)).read().strip()
        )
    except (OSError, ValueError):
        return default


N_DEV = 4
M = 1024
H = M // 2
NSUB = int(os.environ.get("KERNEL_NSUB", "0")) or _read_int_file("kernel_nsub.txt", 4)
Q = H // NSUB
D = 1024
NHOP = N_DEV - 1


def kernel(partial, gamma):
    x = partial[0]
    g = gamma.reshape(1, D)

    def body(x_ref, g_ref, out_ref, xv, xb, recv_r, recv_l,
             dma_sems, ssem_r, rsem_r, ssem_l, rsem_l):
        my = lax.axis_index("i")
        left = (my + N_DEV - 1) % N_DEV
        right = (my + 1) % N_DEV

        comm = _MODE in ("full", "comm")
        comp = _MODE in ("full", "compute")
        stage = _MODE != "barrier"

        cs = [
            [(my + N_DEV - 1 - h) % N_DEV for h in range(NHOP)] + [my],
            [(my + 1 + h) % N_DEV for h in range(NHOP)] + [my],
        ]

        def half_base(direction, idx):
            return cs[direction][idx] * M + direction * H

        def dma_half(direction, idx):
            base = half_base(direction, idx)
            return pltpu.make_async_copy(
                x_ref.at[pl.ds(base, H), :],
                xv.at[pl.ds(base, H), :],
                dma_sems.at[direction, idx],
            )

        def cast_half(direction, idx):
            dma_half(direction, idx).wait()
            base = half_base(direction, idx)
            xb[pl.ds(base, H), :] = xv[pl.ds(base, H), :].astype(jnp.bfloat16)

        def xb_sub(direction, idx, sub):
            return xb[pl.ds(half_base(direction, idx) + sub * Q, Q), :]

        def mk(direction, h, sub):
            if direction == 0:
                buf_r, sem_s, sem_r, tgt = recv_r, ssem_r, rsem_r, right
            else:
                buf_r, sem_s, sem_r, tgt = recv_l, ssem_l, rsem_l, left
            if h == 0:
                src = xb.at[pl.ds(half_base(direction, 0) + sub * Q, Q)]
            else:
                src = buf_r.at[h - 1, sub]
            return pltpu.make_async_remote_copy(
                src_ref=src,
                dst_ref=buf_r.at[h, sub],
                send_sem=sem_s.at[h, sub],
                recv_sem=sem_r.at[h, sub],
                device_id=(tgt,),
                device_id_type=pl.DeviceIdType.MESH,
            )

        if stage:
            for idx in range(N_DEV):
                dma_half(0, idx).start()
                dma_half(1, idx).start()

        if _MODE != "compute":
            barrier_sem = pltpu.get_barrier_semaphore()
            for nbr in (left, right):
                pl.semaphore_signal(
                    barrier_sem, inc=1,
                    device_id=(nbr,), device_id_type=pl.DeviceIdType.MESH,
                )
            pl.semaphore_wait(barrier_sem, 2)

        if stage:
            cast_half(0, 0)
            cast_half(1, 0)
        if comm:
            for sub in range(NSUB):
                mk(0, 0, sub).start()
                mk(1, 0, sub).start()

        if stage:
            for idx in (1, 2, NHOP):
                cast_half(0, idx)
                cast_half(1, idx)

        for h in range(1, NHOP):
            for sub in range(NSUB):
                if comm:
                    mk(0, h - 1, sub).wait_recv()
                if comp:
                    recv_r[h - 1, sub, :, :] = (
                        recv_r[h - 1, sub, :, :] + xb_sub(0, h, sub)
                    )
                if comm:
                    mk(1, h - 1, sub).wait_recv()
                if comp:
                    recv_l[h - 1, sub, :, :] = (
                        recv_l[h - 1, sub, :, :] + xb_sub(1, h, sub)
                    )
                if comm:
                    mk(0, h, sub).start()
                    mk(1, h, sub).start()

        for sub in range(NSUB):
            if comm:
                mk(0, NHOP - 1, sub).wait_recv()
            if comp:
                yr = (
                    recv_r[NHOP - 1, sub, :, :] + xb_sub(0, NHOP, sub)
                ).astype(jnp.float32)
                ms = jnp.mean(yr * yr, axis=-1, keepdims=True)
                out_ref[pl.ds(sub * Q, Q), :] = (
                    yr * lax.rsqrt(ms + 1e-6) * g_ref[:, :]
                )

            if comm:
                mk(1, NHOP - 1, sub).wait_recv()
            if comp:
                yl = (
                    recv_l[NHOP - 1, sub, :, :] + xb_sub(1, NHOP, sub)
                ).astype(jnp.float32)
                ms = jnp.mean(yl * yl, axis=-1, keepdims=True)
                out_ref[pl.ds(H + sub * Q, Q), :] = (
                    yl * lax.rsqrt(ms + 1e-6) * g_ref[:, :]
                )

        if comm:
            for h in range(NHOP):
                for sub in range(NSUB):
                    mk(0, h, sub).wait_send()
                    mk(1, h, sub).wait_send()

    return pl.pallas_call(
        body,
        out_shape=jax.ShapeDtypeStruct((M, D), jnp.float32),
        in_specs=[
            pl.BlockSpec(memory_space=pltpu.MemorySpace.HBM),
            pl.BlockSpec(memory_space=pltpu.VMEM),
        ],
        out_specs=pl.BlockSpec(memory_space=pltpu.VMEM),
        scratch_shapes=[
            pltpu.VMEM((N_DEV * M, D), jnp.float32),
            pltpu.VMEM((N_DEV * M, D), jnp.bfloat16),
            pltpu.VMEM((NHOP, NSUB, Q, D), jnp.bfloat16),
            pltpu.VMEM((NHOP, NSUB, Q, D), jnp.bfloat16),
            pltpu.SemaphoreType.DMA((2, N_DEV)),
            pltpu.SemaphoreType.DMA((NHOP, NSUB)),
            pltpu.SemaphoreType.DMA((NHOP, NSUB)),
            pltpu.SemaphoreType.DMA((NHOP, NSUB)),
            pltpu.SemaphoreType.DMA((NHOP, NSUB)),
        ],
        compiler_params=(
            pltpu.CompilerParams(collective_id=0)
            if _MODE != "compute"
            else pltpu.CompilerParams()
        ),
    )(x, g)


# device time: 10099 ns/iter; 1.1503x vs baseline; 1.1503x over previous
import os

import jax
import jax.numpy as jnp
from jax import lax
from jax.experimental import pallas as pl
from jax.experimental.pallas import tpu as pltpu

_MODE = os.environ.get("KERNEL_MODE", "")
if not _MODE:
    try:
        _MODE = (
            open(os.path.join(os.path.dirname(__file__), "kernel_mode.txt"))
            .read()
            .strip()
        )
    except OSError:
        _MODE = ""
_MODE = _MODE or "full"


def _read_int_file(name, default):
    try:
        return int(
            open(os.path.join(os.path.dirname(__file__), name)).read().strip()
        )
    except (OSError, ValueError):
        return default


N_DEV = 4
M = 1024
H = M // 2
NSUB = int(os.environ.get("KERNEL_NSUB", "0")) or _read_int_file("kernel_nsub.txt", 4)
Q = H // NSUB
D = 1024
NHOP = N_DEV - 1


def kernel(partial, gamma):
    x = partial[0]
    g = gamma.reshape(1, D)

    def body(x_ref, g_ref, out_ref, xv, xb, recv_r, recv_l,
             dma_sems, ssem_r, rsem_r, ssem_l, rsem_l):
        my = lax.axis_index("i")
        left = (my + N_DEV - 1) % N_DEV
        right = (my + 1) % N_DEV

        comm = _MODE in ("full", "comm")
        comp = _MODE in ("full", "compute")
        stage = _MODE != "barrier"

        cs = [
            [(my + N_DEV - 1 - h) % N_DEV for h in range(NHOP)] + [my],
            [(my + 1 + h) % N_DEV for h in range(NHOP)] + [my],
        ]

        def half_base(direction, idx):
            return cs[direction][idx] * M + direction * H

        def dma_half(direction, idx):
            base = half_base(direction, idx)
            return pltpu.make_async_copy(
                x_ref.at[pl.ds(base, H), :],
                xv.at[pl.ds(base, H), :],
                dma_sems.at[direction, idx],
            )

        def cast_half(direction, idx):
            dma_half(direction, idx).wait()
            base = half_base(direction, idx)
            xb[pl.ds(base, H), :] = xv[pl.ds(base, H), :].astype(jnp.bfloat16)

        def xb_sub(direction, idx, sub):
            return xb[pl.ds(half_base(direction, idx) + sub * Q, Q), :]

        def mk(direction, h, sub):
            if direction == 0:
                buf_r, sem_s, sem_r, tgt = recv_r, ssem_r, rsem_r, right
            else:
                buf_r, sem_s, sem_r, tgt = recv_l, ssem_l, rsem_l, left
            if h == 0:
                src = xb.at[pl.ds(half_base(direction, 0) + sub * Q, Q)]
            else:
                src = buf_r.at[h - 1, sub]
            return pltpu.make_async_remote_copy(
                src_ref=src,
                dst_ref=buf_r.at[h, sub],
                send_sem=sem_s.at[h, sub],
                recv_sem=sem_r.at[h, sub],
                device_id=(tgt,),
                device_id_type=pl.DeviceIdType.MESH,
            )

        if stage:
            for idx in range(N_DEV):
                dma_half(0, idx).start()
                dma_half(1, idx).start()

        if _MODE not in ("compute", "empty"):
            barrier_sem = pltpu.get_barrier_semaphore()
            for nbr in (left, right):
                pl.semaphore_signal(
                    barrier_sem, inc=1,
                    device_id=(nbr,), device_id_type=pl.DeviceIdType.MESH,
                )
            pl.semaphore_wait(barrier_sem, 2)

        if stage:
            cast_half(0, 0)
            cast_half(1, 0)
        if comm:
            for sub in range(NSUB):
                mk(0, 0, sub).start()
                mk(1, 0, sub).start()

        if stage:
            for idx in (1, 2, NHOP):
                cast_half(0, idx)
                cast_half(1, idx)

        for h in range(1, NHOP):
            for sub in range(NSUB):
                if comm:
                    mk(0, h - 1, sub).wait_recv()
                if comp:
                    recv_r[h - 1, sub, :, :] = (
                        recv_r[h - 1, sub, :, :] + xb_sub(0, h, sub)
                    )
                if comm:
                    mk(1, h - 1, sub).wait_recv()
                if comp:
                    recv_l[h - 1, sub, :, :] = (
                        recv_l[h - 1, sub, :, :] + xb_sub(1, h, sub)
                    )
                if comm:
                    mk(0, h, sub).start()
                    mk(1, h, sub).start()

        for sub in range(NSUB):
            if comm:
                mk(0, NHOP - 1, sub).wait_recv()
            if comp:
                yr = (
                    recv_r[NHOP - 1, sub, :, :] + xb_sub(0, NHOP, sub)
                ).astype(jnp.float32)
                ms = jnp.mean(yr * yr, axis=-1, keepdims=True)
                out_ref[pl.ds(sub * Q, Q), :] = (
                    yr * lax.rsqrt(ms + 1e-6) * g_ref[:, :]
                )

            if comm:
                mk(1, NHOP - 1, sub).wait_recv()
            if comp:
                yl = (
                    recv_l[NHOP - 1, sub, :, :] + xb_sub(1, NHOP, sub)
                ).astype(jnp.float32)
                ms = jnp.mean(yl * yl, axis=-1, keepdims=True)
                out_ref[pl.ds(H + sub * Q, Q), :] = (
                    yl * lax.rsqrt(ms + 1e-6) * g_ref[:, :]
                )

        if comm:
            for h in range(NHOP):
                for sub in range(NSUB):
                    mk(0, h, sub).wait_send()
                    mk(1, h, sub).wait_send()

    return pl.pallas_call(
        body,
        out_shape=jax.ShapeDtypeStruct((M, D), jnp.float32),
        in_specs=[
            pl.BlockSpec(memory_space=pltpu.MemorySpace.HBM),
            pl.BlockSpec(memory_space=pltpu.VMEM),
        ],
        out_specs=pl.BlockSpec(memory_space=pltpu.VMEM),
        scratch_shapes=[
            pltpu.VMEM((N_DEV * M, D), jnp.float32),
            pltpu.VMEM((N_DEV * M, D), jnp.bfloat16),
            pltpu.VMEM((NHOP, NSUB, Q, D), jnp.bfloat16),
            pltpu.VMEM((NHOP, NSUB, Q, D), jnp.bfloat16),
            pltpu.SemaphoreType.DMA((2, N_DEV)),
            pltpu.SemaphoreType.DMA((NHOP, NSUB)),
            pltpu.SemaphoreType.DMA((NHOP, NSUB)),
            pltpu.SemaphoreType.DMA((NHOP, NSUB)),
            pltpu.SemaphoreType.DMA((NHOP, NSUB)),
        ],
        compiler_params=(
            pltpu.CompilerParams(collective_id=0)
            if _MODE not in ("compute", "empty")
            else pltpu.CompilerParams()
        ),
    )(x, g)
